# Initial kernel scaffold; baseline (speedup 1.0000x reference)
import jax
import jax.numpy as jnp
from jax.experimental import pallas as pl


def kernel(x, edge_index, W1, b1, Wl, bl, Wr):
    raise NotImplementedError("write your pallas kernel here")


# speedup vs baseline: 6.9844x; 6.9844x over previous
import functools

import jax
import jax.numpy as jnp
from jax import lax
from jax.experimental import pallas as pl
from jax.experimental.pallas import tpu as pltpu
from jax.experimental.pallas import tpu_sc as plsc

N_NODES = 10000
N_EDGES = 320000
HID = 128

NC = 2
NS = 16
NW = NC * NS
CHUNK = 128
N_CHUNKS = N_EDGES // CHUNK
CHUNKS_PER_W = N_CHUNKS // NW
CHUNK_REM = N_CHUNKS - CHUNKS_PER_W * NW
N_PAD = 10240
ROWS_PER_TILE = N_PAD // NS
WB = 64
DEG_WB = 128
DEG_W = 16

_f32 = jnp.float32


def _enc_body(x_ref, w1t_ref, b1_ref, wrt_ref, h_ref, hr_ref):
    xb = x_ref[...]
    h = jnp.maximum(
        jnp.dot(xb, w1t_ref[...], preferred_element_type=_f32) + b1_ref[...], 0.0
    )
    h_ref[...] = h
    hr_ref[...] = jnp.dot(h, wrt_ref[...], preferred_element_type=_f32)


def _encode(x, w1t, b1_2d, wrt, blk=1000):
    n = x.shape[0]
    grid = (n // blk,)
    return pl.pallas_call(
        _enc_body,
        grid=grid,
        in_specs=[
            pl.BlockSpec((blk, HID), lambda i: (i, 0)),
            pl.BlockSpec((HID, HID), lambda i: (0, 0)),
            pl.BlockSpec((1, HID), lambda i: (0, 0)),
            pl.BlockSpec((HID, HID), lambda i: (0, 0)),
        ],
        out_specs=[
            pl.BlockSpec((blk, HID), lambda i: (i, 0)),
            pl.BlockSpec((blk, HID), lambda i: (i, 0)),
        ],
        out_shape=[
            jax.ShapeDtypeStruct((n, HID), _f32),
            jax.ShapeDtypeStruct((n, HID), _f32),
        ],
    )(x, w1t, b1_2d, wrt)


def _sc_segment_sum(h, src, dst, zrows, zdeg, ones):
    mesh = plsc.VectorSubcoreMesh(core_axis_name="c", subcore_axis_name="s")

    @functools.partial(
        pl.kernel,
        mesh=mesh,
        out_type=[
            jax.ShapeDtypeStruct((NC, N_PAD, HID), _f32),
            jax.ShapeDtypeStruct((NC, N_PAD, DEG_W), _f32),
        ],
        scratch_types=[
            pltpu.VMEM((CHUNK,), jnp.int32),
            pltpu.VMEM((CHUNK,), jnp.int32),
            pltpu.VMEM((CHUNK, HID), _f32),
            pltpu.VMEM((CHUNK, DEG_W), _f32),
            pltpu.VMEM((WB, HID), _f32),
            pltpu.VMEM((DEG_WB, DEG_W), _f32),
            pltpu.VMEM_SHARED((N_PAD, HID), _f32),
            pltpu.VMEM_SHARED((N_PAD, DEG_W), _f32),
            pltpu.SemaphoreType.DMA,
        ],
        compiler_params=pltpu.CompilerParams(use_tc_tiling_on_sc=False),
    )
    def sc_kernel(
        h_hbm, src_hbm, dst_hbm, zrows_hbm, zdeg_hbm, ones_hbm,
        agg_out, deg_out,
        src_v, dst_v, rows_v, ones_v, bounce_v, degb_v, agg_sh, deg_sh, sem,
    ):
        cid = lax.axis_index("c")
        sid = lax.axis_index("s")
        wid = sid * NC + cid

        base_row = sid * ROWS_PER_TILE
        pltpu.sync_copy(zrows_hbm, bounce_v)
        for k in range(ROWS_PER_TILE // WB):
            pltpu.sync_copy(bounce_v, agg_sh.at[pl.ds(base_row + k * WB, WB)])
        pltpu.sync_copy(zdeg_hbm, degb_v)
        for k in range(ROWS_PER_TILE // DEG_WB):
            pltpu.sync_copy(degb_v, deg_sh.at[pl.ds(base_row + k * DEG_WB, DEG_WB)])
        pltpu.sync_copy(ones_hbm, ones_v)
        plsc.subcore_barrier()

        start = wid * CHUNKS_PER_W + jnp.minimum(wid, CHUNK_REM)
        n_chunks = CHUNKS_PER_W + jnp.where(wid < CHUNK_REM, 1, 0)

        def body(i, carry):
            off = pl.multiple_of((start + i) * CHUNK, CHUNK)
            pltpu.sync_copy(src_hbm.at[pl.ds(off, CHUNK)], src_v)
            pltpu.sync_copy(dst_hbm.at[pl.ds(off, CHUNK)], dst_v)
            pltpu.async_copy(h_hbm.at[src_v], rows_v, sem).wait()
            pltpu.sync_copy(rows_v, agg_sh.at[dst_v], add=True)
            pltpu.sync_copy(ones_v, deg_sh.at[dst_v], add=True)
            return carry

        lax.fori_loop(0, n_chunks, body, 0)
        plsc.subcore_barrier()

        for k in range(ROWS_PER_TILE // WB):
            r0 = base_row + k * WB
            pltpu.sync_copy(agg_sh.at[pl.ds(r0, WB)], bounce_v)
            pltpu.sync_copy(bounce_v, agg_out.at[cid, pl.ds(r0, WB)])
        for k in range(ROWS_PER_TILE // DEG_WB):
            r0 = base_row + k * DEG_WB
            pltpu.sync_copy(deg_sh.at[pl.ds(r0, DEG_WB)], degb_v)
            pltpu.sync_copy(degb_v, deg_out.at[cid, pl.ds(r0, DEG_WB)])

    return sc_kernel(h, src, dst, zrows, zdeg, ones)


def _combine_body(aggp_ref, degp_ref, hr_ref, wlt_ref, bl_ref, out_ref):
    a = aggp_ref[0] + aggp_ref[1]
    d = degp_ref[0, :, 0] + degp_ref[1, :, 0]
    am = a / jnp.maximum(d, 1.0)[:, None]
    out_ref[...] = (
        jnp.dot(am, wlt_ref[...], preferred_element_type=_f32)
        + bl_ref[...]
        + hr_ref[...]
    )


def _combine(aggp, degp, hr, wlt, bl_2d, blk=1000):
    n = hr.shape[0]
    grid = (n // blk,)
    return pl.pallas_call(
        _combine_body,
        grid=grid,
        in_specs=[
            pl.BlockSpec((NC, blk, HID), lambda i: (0, i, 0)),
            pl.BlockSpec((NC, blk, DEG_W), lambda i: (0, i, 0)),
            pl.BlockSpec((blk, HID), lambda i: (i, 0)),
            pl.BlockSpec((HID, HID), lambda i: (0, 0)),
            pl.BlockSpec((1, HID), lambda i: (0, 0)),
        ],
        out_specs=pl.BlockSpec((blk, HID), lambda i: (i, 0)),
        out_shape=jax.ShapeDtypeStruct((n, HID), _f32),
    )(aggp, degp, hr, wlt, bl_2d)


@jax.jit
def kernel(x, edge_index, W1, b1, Wl, bl, Wr):
    ei = edge_index.astype(jnp.int32)
    src = ei[0]
    dst = ei[1]

    h, hr = _encode(x, W1.T, b1.reshape(1, HID), Wr.T)

    zrows = jnp.zeros((WB, HID), _f32)
    zdeg = jnp.zeros((DEG_WB, DEG_W), _f32)
    ones = jnp.ones((CHUNK, DEG_W), _f32)
    aggp, degp = _sc_segment_sum(h, src, dst, zrows, zdeg, ones)

    return _combine(aggp, degp, hr, Wl.T, bl.reshape(1, HID))
